# baseline (speedup 1.0000x reference)
import functools

import jax
import jax.numpy as jnp
from jax import lax
from jax.experimental import pallas as pl
from jax.experimental.pallas import tpu as pltpu
from jax.experimental.pallas import tpu_sc as plsc

D, H, BS, V = 5, 3, 32, 100001
VN = V - 1
CH = 10000
NCHUNK = VN // CH
NVREG = CH // 16
SUB = 25
NSUB = NVREG // SUB
VP = 100352
NC, NS, L = 2, 16, 16


def _spf(x):
    return jnp.full((L,), x, dtype=jnp.float32)


def _spi(x):
    return jnp.full((L,), x, dtype=jnp.int32)


def _sc_body(lg, st_hbm, u_hbm, out_hbm,
             tsb0, tsb1, spb, semA0, semA1, semB0, semB1,
             stv, uv, eosv, stage):
    sid = lax.axis_index("s")
    wid = sid * NC + lax.axis_index("c")
    r = wid
    pltpu.sync_copy(st_hbm, stv)
    pltpu.sync_copy(u_hbm, uv)
    iota = lax.iota(jnp.int32, L)
    zf = jnp.zeros((L,), jnp.float32)
    zi = jnp.zeros((L,), jnp.int32)
    r_lane = iota == _spi(r & 15)
    r_low = _spi(r) < 16
    tsbs = (tsb0, tsb1)
    semsA = (semA0, semA1)
    semsB = (semB0, semB1)

    def rowbase(d, h):
        return pl.multiple_of(((d * H + h) * BS + r) * VP, 8)

    def a_start(d, c, par):
        off = pl.multiple_of(c * CH, 8)
        for h in range(H):
            pltpu.async_copy(lg.at[pl.ds(rowbase(d, h) + off, CH)],
                             spb.at[sid, par, pl.ds(h * CH, CH)], semsA[par])

    def a_drain(d, par):
        for h in range(H):
            pltpu.make_async_copy(lg.at[pl.ds(rowbase(d, h), CH)],
                                  spb.at[sid, par, pl.ds(h * CH, CH)],
                                  semsA[par]).wait()

    def b_start(par):
        pltpu.async_copy(spb.at[sid, par], tsbs[par], semsB[par])

    def b_drain(par):
        pltpu.make_async_copy(spb.at[sid, par], tsbs[par], semsB[par]).wait()

    def item_body(d, carry0):
        doff = pl.multiple_of(d * BS, 8)
        st0 = stv[pl.ds(doff, L)]
        st1 = stv[pl.ds(doff + L, L)]
        u0 = uv[pl.ds(doff, L)]
        u1 = uv[pl.ds(doff + L, L)]
        st_sc = jnp.max(jnp.where(r_lane, jnp.where(r_low, st0, st1), zi))
        u_v = _spf(jnp.max(jnp.where(r_lane, jnp.where(r_low, u0, u1), zf)))
        sel_a = _spi(st_sc) == 0
        sel_b = _spi(st_sc) == 1

        a_start(d, 0, 0)
        a_drain(d, 0)
        b_start(0)
        a_start(d, 1, 1)

        def compute_chunk(c, par, carry):
            sa, sb, sc, sac, sacw, sabc, sabcw, sbb, sabcb, a0, prev_v = carry
            b_drain(par)

            @pl.when(c + 1 < NCHUNK)
            def _():
                a_drain(d, 1 - par)
                b_start(1 - par)

            @pl.when(c + 2 < NCHUNK)
            def _():
                a_start(d, c + 2, par)

            buf = tsbs[par]

            def vec_body(i, ic):
                sa, sb, sc, sac, sacw, sabc, sabcw, sbb, sabcb = ic
                va = buf[pl.ds(i * L, L)]
                vb = buf[pl.ds(CH + i * L, L)]
                vc = buf[pl.ds(2 * CH + i * L, L)]
                ea = jnp.exp(va)
                eb = jnp.exp(vb)
                ec = jnp.exp(vc)
                eac = ea * ec
                eabc = eac * eb
                apc = va + vc
                sa = sa + ea
                sb = sb + eb
                sc = sc + ec
                sac = sac + eac
                sabc = sabc + eabc
                sacw = sacw + eac * apc
                sabcw = sabcw + eabc * apc
                sbb = sbb + eb * vb
                sabcb = sabcb + eabc * vb
                return (sa, sb, sc, sac, sacw, sabc, sabcw, sbb, sabcb)

            res = lax.fori_loop(
                0, NVREG, vec_body,
                (sa, sb, sc, sac, sacw, sabc, sabcw, sbb, sabcb))
            sa, sb, sc, sac, sacw, sabc, sabcw, sbb, sabcb = res
            tot_v = _spf(jnp.sum(jnp.where(sel_a, sa, jnp.where(sel_b, sb, sc))))
            a0 = a0 + jnp.where(iota == _spi(c), tot_v - prev_v, zf)
            prev_v2 = tot_v
            return (sa, sb, sc, sac, sacw, sabc, sabcw, sbb, sabcb, a0, prev_v2)

        def cc_body(cc, carry):
            carry = compute_chunk(cc * 2, 0, carry)
            carry = compute_chunk(cc * 2 + 1, 1, carry)
            return carry

        moments = lax.fori_loop(0, NCHUNK // 2, cc_body, (zf,) * 11)
        sa, sb, sc, sac, sacw, sabc, sabcw, sbb, sabcb, a0, _pv = moments

        for h in range(H):
            eosv[pl.ds(h * L, L)] = zf
        for h in range(H):
            pltpu.sync_copy(lg.at[pl.ds(rowbase(d, h) + V - 1, 1)],
                            eosv.at[pl.ds(h * L, 1)])
        vae = eosv[pl.ds(0, L)]
        vbe = eosv[pl.ds(L, L)]
        vce = eosv[pl.ds(2 * L, L)]
        m0 = iota == 0
        ea = jnp.exp(vae)
        eb = jnp.exp(vbe)
        ec = jnp.exp(vce)
        eac = ea * ec
        eabc = eac * eb
        apc = vae + vce
        sa = sa + jnp.where(m0, ea, zf)
        sb = sb + jnp.where(m0, eb, zf)
        sc = sc + jnp.where(m0, ec, zf)
        sac = sac + jnp.where(m0, eac, zf)
        sabc = sabc + jnp.where(m0, eabc, zf)
        sacw = sacw + jnp.where(m0, eac * apc, zf)
        sabcw = sabcw + jnp.where(m0, eabc * apc, zf)
        sbb = sbb + jnp.where(m0, eb * vbe, zf)
        sabcb = sabcb + jnp.where(m0, eabc * vbe, zf)

        t_v = u_v * _spf(jnp.sum(a0))
        cp0 = plsc.cumsum(a0)
        n0 = plsc.all_reduce_population_count(cp0 < t_v)
        cstar_v = jnp.minimum(n0, _spi(NCHUNK - 1))
        cstar = jnp.max(cstar_v)
        p_v = _spf(jnp.sum(jnp.where(iota < cstar_v, a0, zf)))

        selbase = pl.multiple_of(((d * H + st_sc) * BS + r) * VP, 8)
        pltpu.sync_copy(
            lg.at[pl.ds(selbase + pl.multiple_of(cstar * CH, 8), CH)],
            spb.at[sid, 0, pl.ds(0, CH)])
        pltpu.sync_copy(spb.at[sid, 0, pl.ds(0, CH)], tsb0.at[pl.ds(0, CH)])

        def suba_body(j, jc):
            s0, s1 = jc

            def acc_body(i, acc):
                return acc + jnp.exp(tsb0[pl.ds((j * SUB + i) * L, L)])

            sv = _spf(jnp.sum(lax.fori_loop(0, SUB, acc_body, zf)))
            jm = iota == _spi(j & 15)
            jv = _spi(j)
            s0 = s0 + jnp.where(jm & (jv < 16), sv, zf)
            s1 = s1 + jnp.where(jm & (jv >= 16), sv, zf)
            return (s0, s1)

        s0, s1 = lax.fori_loop(0, NSUB, suba_body, (zf, zf))
        lt_v = t_v - p_v
        sp0 = plsc.cumsum(s0)
        sp1 = plsc.cumsum(s1) + _spf(jnp.sum(s0))
        j0 = plsc.all_reduce_population_count(sp0 < lt_v)
        j1 = plsc.all_reduce_population_count(sp1 < lt_v)
        jstar_v = jnp.minimum(j0 + j1, _spi(NSUB - 1))
        jstar = jnp.max(jstar_v)
        jm0 = iota < jstar_v
        jm1 = (iota + 16) < jstar_v
        p2_v = p_v + _spf(jnp.sum(jnp.where(jm0, s0, zf) +
                                  jnp.where(jm1, s1, zf)))

        def scan_body(i, carry):
            cnt, carv = carry
            v = jnp.exp(tsb0[pl.ds((jstar * SUB + i) * L, L)])
            csv = plsc.cumsum(v)
            cnt = cnt + plsc.all_reduce_population_count((carv + csv) < t_v)
            carv = carv + _spf(jnp.sum(v))
            return (cnt, carv)

        cnt_v, _ = lax.fori_loop(0, SUB, scan_body, (zi, p2_v))
        nxt = jnp.minimum(cstar * CH + jstar * (SUB * L) + jnp.max(cnt_v),
                          VN - 1)

        base = pl.multiple_of(nxt & (-16), 8)
        lane_m = iota == _spi(nxt - base)
        xn = []
        for h in range(H):
            pltpu.sync_copy(lg.at[pl.ds(rowbase(d, h) + base, L)],
                            tsb1.at[pl.ds(0, L)])
            xn.append(jnp.sum(jnp.where(lane_m, tsb1[pl.ds(0, L)], zf)))

        vals = [jnp.sum(sa), jnp.sum(sb), jnp.sum(sc), jnp.sum(sac),
                jnp.sum(sacw), jnp.sum(sabc), jnp.sum(sabcw), jnp.sum(sbb),
                jnp.sum(sabcb), xn[0], xn[1], xn[2]]
        out_v = zf
        for k, s in enumerate(vals):
            out_v = out_v + jnp.where(iota == k, _spf(s), zf)
        out_v = out_v + jnp.where(iota == 12, _spi(nxt).astype(jnp.float32), zf)
        stage[:] = out_v
        pltpu.sync_copy(stage, out_hbm.at[pl.ds((d * BS + r) * L, L)])
        return carry0

    lax.fori_loop(0, D, item_body, 0)


@functools.lru_cache(maxsize=1)
def _sc_moments():
    mesh = plsc.VectorSubcoreMesh(core_axis_name="c", subcore_axis_name="s",
                                  num_cores=NC, num_subcores=NS)
    return pl.kernel(
        _sc_body,
        out_type=jax.ShapeDtypeStruct((D * BS * L,), jnp.float32),
        mesh=mesh,
        compiler_params=pltpu.CompilerParams(use_tc_tiling_on_sc=False,
                                             needs_layout_passes=False),
        scratch_types=[
            pltpu.VMEM((H * CH,), jnp.float32),
            pltpu.VMEM((H * CH,), jnp.float32),
            pltpu.VMEM_SHARED((NS, 2, H * CH), jnp.float32),
            pltpu.SemaphoreType.DMA,
            pltpu.SemaphoreType.DMA,
            pltpu.SemaphoreType.DMA,
            pltpu.SemaphoreType.DMA,
            pltpu.VMEM((D * BS,), jnp.int32),
            pltpu.VMEM((D * BS,), jnp.float32),
            pltpu.VMEM((H * L,), jnp.float32),
            pltpu.VMEM((L,), jnp.float32),
        ],
    )


def _fmt_body(x_ref, o_ref):
    for rr in range(8):
        o_ref[pl.ds(rr * VP, V)] = x_ref[rr, :]


def _format(x2d):
    return pl.pallas_call(
        _fmt_body,
        grid=(D * H * BS // 8,),
        in_specs=[pl.BlockSpec((8, V), lambda i: (i, 0))],
        out_specs=pl.BlockSpec((8 * VP,), lambda i: (i,)),
        out_shape=jax.ShapeDtypeStruct((D * H * BS * VP,), jnp.float32),
    )(x2d)


def _combine_body(m_ref, st_ref, o_ref):
    total = jnp.zeros((BS,), jnp.float32)
    cum = jnp.ones((BS,), jnp.float32)
    s_cur = jnp.zeros((BS,), jnp.float32)
    for d in range(D):
        m = m_ref[d]
        sa, sb, sc = m[:, 0], m[:, 1], m[:, 2]
        sac, sacw = m[:, 3], m[:, 4]
        sabc, sabcw = m[:, 5], m[:, 6]
        sbb, sabcb = m[:, 7], m[:, 8]
        xan, xbn, xcn = m[:, 9], m[:, 10], m[:, 11]
        st = st_ref[d]
        la, lb, lc = jnp.log(sa), jnp.log(sb), jnp.log(sc)
        nac = sa * sc
        nabc = nac * sb
        sum_ac = sac / nac
        sum_acw = (sacw - (la + lc) * sac) / nac
        sum_abc = sabc / nabc
        sum_abcw = (sabcw - (la + lc) * sabc) / nabc
        sum_bb = (sbb - lb * sb) / sb
        sum_abcb = (sabcb - lb * sabc) / nabc
        middle = s_cur * (sum_ac - sum_abc) + (sum_acw - sum_abcw)
        last = sum_ac * sum_bb - sum_abcb
        total = total + cum * (middle + last)
        lpa, lpb, lpc = xan - la, xbn - lb, xcn - lc
        s_cur = s_cur + lpa + lpb + lpc
        if d < D - 1:
            mult = jnp.where(st == 0, lpb + lpc,
                             jnp.where(st == 1, lpa + lpc, lpa + lpb))
            cum = cum * jnp.exp(mult)
    o_ref[:, :] = jnp.full((1, 1), -jnp.mean(total), dtype=jnp.float32)


def _combine(m, st):
    return pl.pallas_call(
        _combine_body,
        out_shape=jax.ShapeDtypeStruct((1, 1), jnp.float32),
    )(m, st)


def kernel(logits, sampling_target, u):
    st = sampling_target.astype(jnp.int32)
    lgp = _format(logits.reshape(D * H * BS, V))
    m = _sc_moments()(lgp, st.reshape(-1), u.reshape(-1))
    return _combine(m.reshape(D, BS, L), st)[0, 0]


# speedup vs baseline: 1.0645x; 1.0645x over previous
import functools

import jax
import jax.numpy as jnp
from jax import lax
from jax.experimental import pallas as pl
from jax.experimental.pallas import tpu as pltpu
from jax.experimental.pallas import tpu_sc as plsc

D, H, BS, V = 5, 3, 32, 100001
VN = V - 1
CH = 10000
NCHUNK = VN // CH
NVREG = CH // 16
SUB = 25
NSUB = NVREG // SUB
VP = 100352
NC, NS, L = 2, 16, 16


def _spf(x):
    return jnp.full((L,), x, dtype=jnp.float32)


def _spi(x):
    return jnp.full((L,), x, dtype=jnp.int32)


def _sc_body(lg, st_hbm, u_hbm, out_hbm,
             tsb0, tsb1, semA0, semA1,
             stv, uv, eosv, stage):
    wid = lax.axis_index("s") * NC + lax.axis_index("c")
    r = wid
    pltpu.sync_copy(st_hbm, stv)
    pltpu.sync_copy(u_hbm, uv)
    iota = lax.iota(jnp.int32, L)
    zf = jnp.zeros((L,), jnp.float32)
    zi = jnp.zeros((L,), jnp.int32)
    r_lane = iota == _spi(r & 15)
    r_low = _spi(r) < 16
    tsbs = (tsb0, tsb1)
    semsA = (semA0, semA1)

    def rowbase(d, h):
        return pl.multiple_of(((d * H + h) * BS + r) * VP, 8)

    def a_start(d, c, par):
        off = pl.multiple_of(c * CH, 8)
        for h in range(H):
            pltpu.async_copy(lg.at[pl.ds(rowbase(d, h) + off, CH)],
                             tsbs[par].at[pl.ds(h * CH, CH)], semsA[par])

    def a_drain(d, par):
        for h in range(H):
            pltpu.make_async_copy(lg.at[pl.ds(rowbase(d, h), CH)],
                                  tsbs[par].at[pl.ds(h * CH, CH)],
                                  semsA[par]).wait()

    def item_body(d, carry0):
        doff = pl.multiple_of(d * BS, 8)
        st0 = stv[pl.ds(doff, L)]
        st1 = stv[pl.ds(doff + L, L)]
        u0 = uv[pl.ds(doff, L)]
        u1 = uv[pl.ds(doff + L, L)]
        st_sc = jnp.max(jnp.where(r_lane, jnp.where(r_low, st0, st1), zi))
        u_v = _spf(jnp.max(jnp.where(r_lane, jnp.where(r_low, u0, u1), zf)))
        sel_a = _spi(st_sc) == 0
        sel_b = _spi(st_sc) == 1

        a_start(d, 0, 0)
        a_start(d, 1, 1)

        def compute_chunk(c, par, carry):
            sa, sb, sc, sac, sacw, sabc, sabcw, sbb, sabcb, a0 = carry
            a_drain(d, par)
            buf = tsbs[par]

            def vec_body(i, ic):
                sa, sb, sc, sac, sacw, sabc, sabcw, sbb, sabcb, cs = ic
                va = buf[pl.ds(i * L, L)]
                vb = buf[pl.ds(CH + i * L, L)]
                vc = buf[pl.ds(2 * CH + i * L, L)]
                ea = jnp.exp(va)
                eb = jnp.exp(vb)
                ec = jnp.exp(vc)
                eac = ea * ec
                eabc = eac * eb
                apc = va + vc
                sa = sa + ea
                sb = sb + eb
                sc = sc + ec
                sac = sac + eac
                sabc = sabc + eabc
                sacw = sacw + eac * apc
                sabcw = sabcw + eabc * apc
                sbb = sbb + eb * vb
                sabcb = sabcb + eabc * vb
                cs = cs + jnp.where(sel_a, ea, jnp.where(sel_b, eb, ec))
                return (sa, sb, sc, sac, sacw, sabc, sabcw, sbb, sabcb, cs)

            res = lax.fori_loop(
                0, NVREG, vec_body,
                (sa, sb, sc, sac, sacw, sabc, sabcw, sbb, sabcb, zf))
            sa, sb, sc, sac, sacw, sabc, sabcw, sbb, sabcb, cs = res
            a0 = a0 + jnp.where(iota == _spi(c), _spf(jnp.sum(cs)), zf)

            @pl.when(c + 2 < NCHUNK)
            def _():
                a_start(d, c + 2, par)

            return (sa, sb, sc, sac, sacw, sabc, sabcw, sbb, sabcb, a0)

        def cc_body(cc, carry):
            carry = compute_chunk(cc * 2, 0, carry)
            carry = compute_chunk(cc * 2 + 1, 1, carry)
            return carry

        moments = lax.fori_loop(0, NCHUNK // 2, cc_body, (zf,) * 10)
        sa, sb, sc, sac, sacw, sabc, sabcw, sbb, sabcb, a0 = moments

        for h in range(H):
            eosv[pl.ds(h * L, L)] = zf
        for h in range(H):
            pltpu.sync_copy(lg.at[pl.ds(rowbase(d, h) + V - 1, 1)],
                            eosv.at[pl.ds(h * L, 1)])
        vae = eosv[pl.ds(0, L)]
        vbe = eosv[pl.ds(L, L)]
        vce = eosv[pl.ds(2 * L, L)]
        m0 = iota == 0
        ea = jnp.exp(vae)
        eb = jnp.exp(vbe)
        ec = jnp.exp(vce)
        eac = ea * ec
        eabc = eac * eb
        apc = vae + vce
        sa = sa + jnp.where(m0, ea, zf)
        sb = sb + jnp.where(m0, eb, zf)
        sc = sc + jnp.where(m0, ec, zf)
        sac = sac + jnp.where(m0, eac, zf)
        sabc = sabc + jnp.where(m0, eabc, zf)
        sacw = sacw + jnp.where(m0, eac * apc, zf)
        sabcw = sabcw + jnp.where(m0, eabc * apc, zf)
        sbb = sbb + jnp.where(m0, eb * vbe, zf)
        sabcb = sabcb + jnp.where(m0, eabc * vbe, zf)

        t_v = u_v * _spf(jnp.sum(a0))
        cp0 = plsc.cumsum(a0)
        n0 = plsc.all_reduce_population_count(cp0 < t_v)
        cstar_v = jnp.minimum(n0, _spi(NCHUNK - 1))
        cstar = jnp.max(cstar_v)
        p_v = _spf(jnp.sum(jnp.where(iota < cstar_v, a0, zf)))

        selbase = pl.multiple_of(((d * H + st_sc) * BS + r) * VP, 8)
        pltpu.sync_copy(
            lg.at[pl.ds(selbase + pl.multiple_of(cstar * CH, 8), CH)],
            tsb0.at[pl.ds(0, CH)])

        def suba_body(j, jc):
            s0, s1 = jc

            def acc_body(i, acc):
                return acc + jnp.exp(tsb0[pl.ds((j * SUB + i) * L, L)])

            sv = _spf(jnp.sum(lax.fori_loop(0, SUB, acc_body, zf)))
            jm = iota == _spi(j & 15)
            jv = _spi(j)
            s0 = s0 + jnp.where(jm & (jv < 16), sv, zf)
            s1 = s1 + jnp.where(jm & (jv >= 16), sv, zf)
            return (s0, s1)

        s0, s1 = lax.fori_loop(0, NSUB, suba_body, (zf, zf))
        lt_v = t_v - p_v
        sp0 = plsc.cumsum(s0)
        sp1 = plsc.cumsum(s1) + _spf(jnp.sum(s0))
        j0 = plsc.all_reduce_population_count(sp0 < lt_v)
        j1 = plsc.all_reduce_population_count(sp1 < lt_v)
        jstar_v = jnp.minimum(j0 + j1, _spi(NSUB - 1))
        jstar = jnp.max(jstar_v)
        jm0 = iota < jstar_v
        jm1 = (iota + 16) < jstar_v
        p2_v = p_v + _spf(jnp.sum(jnp.where(jm0, s0, zf) +
                                  jnp.where(jm1, s1, zf)))

        def scan_body(i, carry):
            cnt, carv = carry
            v = jnp.exp(tsb0[pl.ds((jstar * SUB + i) * L, L)])
            csv = plsc.cumsum(v)
            cnt = cnt + plsc.all_reduce_population_count((carv + csv) < t_v)
            carv = carv + _spf(jnp.sum(v))
            return (cnt, carv)

        cnt_v, _ = lax.fori_loop(0, SUB, scan_body, (zi, p2_v))
        nxt = jnp.minimum(cstar * CH + jstar * (SUB * L) + jnp.max(cnt_v),
                          VN - 1)

        base = pl.multiple_of(nxt & (-16), 8)
        lane_m = iota == _spi(nxt - base)
        xn = []
        for h in range(H):
            pltpu.sync_copy(lg.at[pl.ds(rowbase(d, h) + base, L)],
                            tsb1.at[pl.ds(0, L)])
            xn.append(jnp.sum(jnp.where(lane_m, tsb1[pl.ds(0, L)], zf)))

        vals = [jnp.sum(sa), jnp.sum(sb), jnp.sum(sc), jnp.sum(sac),
                jnp.sum(sacw), jnp.sum(sabc), jnp.sum(sabcw), jnp.sum(sbb),
                jnp.sum(sabcb), xn[0], xn[1], xn[2]]
        out_v = zf
        for k, s in enumerate(vals):
            out_v = out_v + jnp.where(iota == k, _spf(s), zf)
        out_v = out_v + jnp.where(iota == 12, _spi(nxt).astype(jnp.float32), zf)
        stage[:] = out_v
        pltpu.sync_copy(stage, out_hbm.at[pl.ds((d * BS + r) * L, L)])
        return carry0

    lax.fori_loop(0, D, item_body, 0)


@functools.lru_cache(maxsize=1)
def _sc_moments():
    mesh = plsc.VectorSubcoreMesh(core_axis_name="c", subcore_axis_name="s",
                                  num_cores=NC, num_subcores=NS)
    return pl.kernel(
        _sc_body,
        out_type=jax.ShapeDtypeStruct((D * BS * L,), jnp.float32),
        mesh=mesh,
        compiler_params=pltpu.CompilerParams(use_tc_tiling_on_sc=False,
                                             needs_layout_passes=False),
        scratch_types=[
            pltpu.VMEM((H * CH,), jnp.float32),
            pltpu.VMEM((H * CH,), jnp.float32),
            pltpu.SemaphoreType.DMA,
            pltpu.SemaphoreType.DMA,
            pltpu.VMEM((D * BS,), jnp.int32),
            pltpu.VMEM((D * BS,), jnp.float32),
            pltpu.VMEM((H * L,), jnp.float32),
            pltpu.VMEM((L,), jnp.float32),
        ],
    )


def _fmt_body(x_ref, o_ref):
    for rr in range(8):
        o_ref[pl.ds(rr * VP, V)] = x_ref[rr, :]


def _format(x2d):
    return pl.pallas_call(
        _fmt_body,
        grid=(D * H * BS // 8,),
        in_specs=[pl.BlockSpec((8, V), lambda i: (i, 0))],
        out_specs=pl.BlockSpec((8 * VP,), lambda i: (i,)),
        out_shape=jax.ShapeDtypeStruct((D * H * BS * VP,), jnp.float32),
    )(x2d)


def _combine_body(m_ref, st_ref, o_ref):
    total = jnp.zeros((BS,), jnp.float32)
    cum = jnp.ones((BS,), jnp.float32)
    s_cur = jnp.zeros((BS,), jnp.float32)
    for d in range(D):
        m = m_ref[d]
        sa, sb, sc = m[:, 0], m[:, 1], m[:, 2]
        sac, sacw = m[:, 3], m[:, 4]
        sabc, sabcw = m[:, 5], m[:, 6]
        sbb, sabcb = m[:, 7], m[:, 8]
        xan, xbn, xcn = m[:, 9], m[:, 10], m[:, 11]
        st = st_ref[d]
        la, lb, lc = jnp.log(sa), jnp.log(sb), jnp.log(sc)
        nac = sa * sc
        nabc = nac * sb
        sum_ac = sac / nac
        sum_acw = (sacw - (la + lc) * sac) / nac
        sum_abc = sabc / nabc
        sum_abcw = (sabcw - (la + lc) * sabc) / nabc
        sum_bb = (sbb - lb * sb) / sb
        sum_abcb = (sabcb - lb * sabc) / nabc
        middle = s_cur * (sum_ac - sum_abc) + (sum_acw - sum_abcw)
        last = sum_ac * sum_bb - sum_abcb
        total = total + cum * (middle + last)
        lpa, lpb, lpc = xan - la, xbn - lb, xcn - lc
        s_cur = s_cur + lpa + lpb + lpc
        if d < D - 1:
            mult = jnp.where(st == 0, lpb + lpc,
                             jnp.where(st == 1, lpa + lpc, lpa + lpb))
            cum = cum * jnp.exp(mult)
    o_ref[:, :] = jnp.full((1, 1), -jnp.mean(total), dtype=jnp.float32)


def _combine(m, st):
    return pl.pallas_call(
        _combine_body,
        out_shape=jax.ShapeDtypeStruct((1, 1), jnp.float32),
    )(m, st)


def kernel(logits, sampling_target, u):
    st = sampling_target.astype(jnp.int32)
    lgp = _format(logits.reshape(D * H * BS, V))
    m = _sc_moments()(lgp, st.reshape(-1), u.reshape(-1))
    return _combine(m.reshape(D, BS, L), st)[0, 0]
